# baseline (speedup 1.0000x reference)
import functools
import math

import jax
import jax.numpy as jnp
from jax import lax
from jax.experimental import pallas as pl
from jax.experimental.pallas import tpu as pltpu
from jax.experimental.pallas import tpu_sc as plsc

BATCH = 16384
EMBED = 64
FEAT = 64
HIDDEN = 256
KDIM = EMBED + EMBED + FEAT

NC = 2
NS = 16
NW = NC * NS
B_PER_W = BATCH // NW
CHUNK = 128
NCHUNK = B_PER_W // CHUNK


def _gather_body(uidx_hbm, iidx_hbm, uemb_hbm, iemb_hbm, u_out, i_out,
                 idx_v, rows_u, rows_i, sem):
    wid = lax.axis_index("s") * NC + lax.axis_index("c")
    base = wid * B_PER_W
    pltpu.sync_copy(uidx_hbm.at[wid], idx_v.at[pl.ds(0, NCHUNK)])
    pltpu.sync_copy(iidx_hbm.at[wid], idx_v.at[pl.ds(NCHUNK, NCHUNK)])
    copies = []
    for j in range(NCHUNK):
        copies.append(pltpu.async_copy(
            uemb_hbm.at[idx_v.at[j]], rows_u.at[pl.ds(j * CHUNK, CHUNK)], sem))
    for j in range(NCHUNK):
        copies.append(pltpu.async_copy(
            iemb_hbm.at[idx_v.at[NCHUNK + j]], rows_i.at[pl.ds(j * CHUNK, CHUNK)], sem))
    for c in copies:
        c.wait()
    pltpu.sync_copy(rows_u, u_out.at[pl.ds(base, B_PER_W)])
    pltpu.sync_copy(rows_i, i_out.at[pl.ds(base, B_PER_W)])


@functools.lru_cache(maxsize=None)
def _sc_gather():
    return pl.kernel(
        _gather_body,
        out_type=(jax.ShapeDtypeStruct((BATCH, EMBED), jnp.float32),
                  jax.ShapeDtypeStruct((BATCH, EMBED), jnp.float32)),
        mesh=plsc.VectorSubcoreMesh(core_axis_name="c", subcore_axis_name="s",
                                    num_cores=NC, num_subcores=NS),
        scratch_types=[
            pltpu.VMEM((2 * NCHUNK, CHUNK), jnp.int32),
            pltpu.VMEM((B_PER_W, EMBED), jnp.float32),
            pltpu.VMEM((B_PER_W, EMBED), jnp.float32),
            pltpu.SemaphoreType.DMA,
        ],
        compiler_params=pltpu.CompilerParams(use_tc_tiling_on_sc=False),
    )


ROWS_BLK = 2048
GRID = BATCH // ROWS_BLK


def _mlp_body(u_ref, i_ref, f_ref, w1_ref, b1_ref, wrp_ref, brp_ref,
              rat_ref, play_ref):
    x = (jnp.dot(u_ref[...], w1_ref[0:EMBED, :],
                 preferred_element_type=jnp.float32)
         + jnp.dot(i_ref[...], w1_ref[EMBED:2 * EMBED, :],
                   preferred_element_type=jnp.float32)
         + jnp.dot(f_ref[...], w1_ref[2 * EMBED:KDIM, :],
                   preferred_element_type=jnp.float32)
         + b1_ref[...])
    h = 0.5 * x * (1.0 + lax.erf(x * (1.0 / math.sqrt(2.0))))
    o = jnp.dot(h, wrp_ref[...], preferred_element_type=jnp.float32) + brp_ref[...]
    rat_ref[...] = jax.nn.sigmoid(o[:, 0:1])
    play_ref[...] = jnp.maximum(o[:, 1:2], 0.0)


def _mlp(u_rows, i_rows, feature_input, W1, b1, Wrp, brp, interpret=False):
    return pl.pallas_call(
        _mlp_body,
        grid=(GRID,),
        in_specs=[
            pl.BlockSpec((ROWS_BLK, EMBED), lambda i: (i, 0)),
            pl.BlockSpec((ROWS_BLK, EMBED), lambda i: (i, 0)),
            pl.BlockSpec((ROWS_BLK, FEAT), lambda i: (i, 0)),
            pl.BlockSpec((KDIM, HIDDEN), lambda i: (0, 0)),
            pl.BlockSpec((1, HIDDEN), lambda i: (0, 0)),
            pl.BlockSpec((HIDDEN, 2), lambda i: (0, 0)),
            pl.BlockSpec((1, 2), lambda i: (0, 0)),
        ],
        out_specs=[
            pl.BlockSpec((ROWS_BLK, 1), lambda i: (i, 0)),
            pl.BlockSpec((ROWS_BLK, 1), lambda i: (i, 0)),
        ],
        out_shape=[
            jax.ShapeDtypeStruct((BATCH, 1), jnp.float32),
            jax.ShapeDtypeStruct((BATCH, 1), jnp.float32),
        ],
        interpret=interpret,
    )(u_rows, i_rows, feature_input, W1, b1, Wrp, brp)


def kernel(user_input, item_input, feature_input, user_emb, item_emb,
           W1, b1, Wr, br, Wp, bp):
    uidx = user_input.reshape(NW, NCHUNK, CHUNK)
    iidx = item_input.reshape(NW, NCHUNK, CHUNK)
    u_rows, i_rows = _sc_gather()(uidx, iidx, user_emb, item_emb)
    Wrp = jnp.concatenate([Wr, Wp], axis=1)
    brp = jnp.concatenate([br, bp]).reshape(1, 2)
    rating, playtime = _mlp(u_rows, i_rows, feature_input,
                            W1, b1.reshape(1, HIDDEN), Wrp, brp)
    return (rating, playtime)
